# Initial kernel scaffold; baseline (speedup 1.0000x reference)
import jax
import jax.numpy as jnp
from jax.experimental import pallas as pl


def kernel(x, edge_index, edge_attr1, edge_attr2, batch, params):
    raise NotImplementedError("write your pallas kernel here")


# speedup vs baseline: 1.0922x; 1.0922x over previous
import functools

import jax
import jax.numpy as jnp
from jax.experimental import pallas as pl
from jax.experimental.pallas import tpu as pltpu

_DIN, _D1, _D2, _D3, _D4 = 4, 64, 128, 256, 64
_N, _E, _G, _NEF = 1024, 2048, 32, 3

_LAYERS = (
    ("conv_a", _DIN, _D1, 32),
    ("conv_b", _D1, _D2, 16),
    ("conv_c", _D2, _D3, 8),
    ("conv_d", _D3, _D4, 8),
)


def _conv_body(din, dout, kc, nk, head, *refs):
    if head:
        (h_ref, src, dst, ea1, ea2, w1, b1, w2, b2, w3r, b3r, root, bias,
         batch_row, l1w, l1b, l2w, l2b, l3w, l3b, ow, ob, out_ref,
         h2_scr, xs_scr, msg_scr) = refs
    else:
        (h_ref, src, dst, ea1, ea2, w1, b1, w2, b2, w3r, b3r, root, bias,
         out_ref, h2_scr, xs_scr, msg_scr) = refs
    k = pl.program_id(0)

    @pl.when(k == 0)
    def _init():
        ea = ea1[...] + ea2[...]
        h1 = jnp.maximum(jnp.dot(ea, w1[...], preferred_element_type=jnp.float32, precision=jax.lax.Precision.HIGHEST)
                         + b1[...], 0.0)
        h2 = jnp.maximum(
            jnp.dot(h1, w2[...], preferred_element_type=jnp.float32, precision=jax.lax.Precision.HIGHEST) + b2[...], 0.0)
        for i in range(nk):
            h2_scr[i] = h2[:, i * kc:(i + 1) * kc]
        col_ids = jax.lax.broadcasted_iota(jnp.int32, (_E, _N), 1)
        gat = (src[...] == col_ids).astype(jnp.float32)
        hv = h_ref[...]
        h_hi = hv.astype(jnp.bfloat16).astype(jnp.float32)
        xs = (jnp.dot(gat, h_hi, preferred_element_type=jnp.float32)
              + jnp.dot(gat, hv - h_hi, preferred_element_type=jnp.float32))
        xs_scr[...] = xs
        msg_scr[...] = jnp.dot(xs, b3r[...], preferred_element_type=jnp.float32, precision=jax.lax.Precision.HIGHEST)

    h2c = h2_scr[k]
    xs = xs_scr[...]
    w3 = w3r[...]
    msg = msg_scr[...]
    for j in range(kc):
        msg = msg + jnp.dot(h2c[:, j:j + 1] * xs, w3[j * din:(j + 1) * din, :],
                            preferred_element_type=jnp.float32, precision=jax.lax.Precision.HIGHEST)
    msg_scr[...] = msg

    @pl.when(k == nk - 1)
    def _fini():
        row_ids = jax.lax.broadcasted_iota(jnp.int32, (_N, _E), 0)
        sca = (dst[...] == row_ids).astype(jnp.float32)
        mv = msg_scr[...]
        m_hi = mv.astype(jnp.bfloat16).astype(jnp.float32)
        agg = (jnp.dot(sca, m_hi, preferred_element_type=jnp.float32)
               + jnp.dot(sca, mv - m_hi, preferred_element_type=jnp.float32))
        h = jnp.maximum(
            agg + jnp.dot(h_ref[...], root[...], preferred_element_type=jnp.float32, precision=jax.lax.Precision.HIGHEST)
            + bias[...], 0.0)
        if not head:
            out_ref[...] = h
        else:
            h = jnp.dot(h, l1w[...], preferred_element_type=jnp.float32, precision=jax.lax.Precision.HIGHEST) + l1b[...]
            h = jnp.dot(h, l2w[...], preferred_element_type=jnp.float32, precision=jax.lax.Precision.HIGHEST) + l2b[...]
            h = jnp.dot(h, l3w[...], preferred_element_type=jnp.float32, precision=jax.lax.Precision.HIGHEST) + l3b[...]
            g_ids = jax.lax.broadcasted_iota(jnp.int32, (_G, _N), 0)
            starts = jnp.sum((batch_row[...] < g_ids).astype(jnp.int32),
                             axis=1, keepdims=True)
            n_ids = jax.lax.broadcasted_iota(jnp.int32, (_G, _N), 1)
            out = ob[...]
            for t in range(3):
                sel = (n_ids == starts + t).astype(jnp.float32)
                ft = jnp.dot(sel, h, preferred_element_type=jnp.float32, precision=jax.lax.Precision.HIGHEST)
                out = out + jnp.dot(ft, ow[...][64 * t:64 * (t + 1), :],
                                    preferred_element_type=jnp.float32, precision=jax.lax.Precision.HIGHEST)
            out_ref[...] = out


def _full(shape):
    return pl.BlockSpec(shape, lambda k: (0, 0))


def kernel(x, edge_index, edge_attr1, edge_attr2, batch, params):
    src = edge_index[0].reshape(_E, 1)
    dst = edge_index[1].reshape(1, _E)
    batch_row = batch.reshape(1, _N)
    h = x
    for (name, din, dout, kc) in _LAYERS:
        p = params[name]
        m = p["mlp"]
        w3r = m["W3"].reshape(64, din, dout).reshape(64 * din, dout)
        nk = 64 // kc
        head = name == "conv_d"
        ops = [h, src, dst, edge_attr1, edge_attr2,
               m["W1"], m["b1"].reshape(1, 64), m["W2"], m["b2"].reshape(1, 64),
               w3r, m["b3"].reshape(din, dout), p["root"],
               p["bias"].reshape(1, dout)]
        in_specs = [
            _full((_N, din)), _full((_E, 1)), _full((1, _E)),
            _full((_E, _NEF)), _full((_E, _NEF)),
            _full((_NEF, 64)), _full((1, 64)), _full((64, 64)), _full((1, 64)),
            pl.BlockSpec((kc * din, dout), lambda k: (k, 0)),
            _full((din, dout)), _full((din, dout)), _full((1, dout)),
        ]
        if head:
            ops += [batch_row, params["lin1W"], params["lin1b"].reshape(1, 128),
                    params["lin2W"], params["lin2b"].reshape(1, 64),
                    params["lin3W"], params["lin3b"].reshape(1, 64),
                    params["outW"], params["outb"].reshape(1, 1)]
            in_specs += [_full((1, _N)), _full((_D4, 128)), _full((1, 128)),
                         _full((128, 64)), _full((1, 64)), _full((64, 64)),
                         _full((1, 64)), _full((192, 1)), _full((1, 1))]
            out_shape = jax.ShapeDtypeStruct((_G, 1), jnp.float32)
            out_specs = _full((_G, 1))
        else:
            out_shape = jax.ShapeDtypeStruct((_N, dout), jnp.float32)
            out_specs = _full((_N, dout))
        h = pl.pallas_call(
            functools.partial(_conv_body, din, dout, kc, nk, head),
            grid=(nk,),
            in_specs=in_specs,
            out_specs=out_specs,
            out_shape=out_shape,
            scratch_shapes=[pltpu.VMEM((nk, _E, kc), jnp.float32),
                            pltpu.VMEM((_E, din), jnp.float32),
                            pltpu.VMEM((_E, dout), jnp.float32)],
        )(*ops)
    return h


# speedup vs baseline: 1.3939x; 1.2763x over previous
import functools

import jax
import jax.numpy as jnp
from jax.experimental import pallas as pl
from jax.experimental.pallas import tpu as pltpu

_DIN, _D1, _D2, _D3, _D4 = 4, 64, 128, 256, 64
_N, _E, _G, _NEF = 1024, 2048, 32, 3

_HIGH = jax.lax.Precision.HIGHEST

_LAYERS = (
    ("conv_a", _DIN, _D1, 64),
    ("conv_b", _D1, _D2, 16),
    ("conv_c", _D2, _D3, 8),
    ("conv_d", _D3, _D4, 8),
)


def _dot(a, b, precision=_HIGH):
    return jnp.dot(a, b, preferred_element_type=jnp.float32, precision=precision)


def _onehot_matmul(onehot, dense):
    hi = dense.astype(jnp.bfloat16).astype(jnp.float32)
    return (jnp.dot(onehot, hi, preferred_element_type=jnp.float32)
            + jnp.dot(onehot, dense - hi, preferred_element_type=jnp.float32))


def _conv_body(din, dout, kc, nk, head, *refs):
    if head:
        (h_ref, src, dst, ea1, ea2, w1, b1, w2, b2, w3c, b3r, root, bias,
         batch_row, l1w, l1b, l2w, l2b, l3w, l3b, ow, ob, out_ref,
         h2_scr, xs_scr, msg_scr) = refs
    else:
        (h_ref, src, dst, ea1, ea2, w1, b1, w2, b2, w3c, b3r, root, bias,
         out_ref, h2_scr, xs_scr, msg_scr) = refs
    k = pl.program_id(0)

    @pl.when(k == 0)
    def _init():
        ea = ea1[...] + ea2[...]
        h1 = jnp.maximum(_dot(ea, w1[...]) + b1[...], 0.0)
        h2 = jnp.maximum(_dot(h1, w2[...]) + b2[...], 0.0)
        for i in range(nk):
            h2_scr[i] = h2[:, i * kc:(i + 1) * kc]
        col_ids = jax.lax.broadcasted_iota(jnp.int32, (_E, _N), 1)
        gat = (src[...] == col_ids).astype(jnp.float32)
        xs = _onehot_matmul(gat, h_ref[...])
        xs_scr[...] = xs
        msg_scr[...] = _dot(xs, b3r[...])

    h2c = h2_scr[k]
    xs = xs_scr[...]
    u = _dot(xs, w3c[...])
    msg = msg_scr[...]
    for j in range(kc):
        msg = msg + h2c[:, j:j + 1] * u[:, j * dout:(j + 1) * dout]
    msg_scr[...] = msg

    @pl.when(k == nk - 1)
    def _fini():
        row_ids = jax.lax.broadcasted_iota(jnp.int32, (_N, _E), 0)
        sca = (dst[...] == row_ids).astype(jnp.float32)
        agg = _onehot_matmul(sca, msg_scr[...])
        h = jnp.maximum(agg + _dot(h_ref[...], root[...]) + bias[...], 0.0)
        if not head:
            out_ref[...] = h
        else:
            h = _dot(h, l1w[...]) + l1b[...]
            h = _dot(h, l2w[...]) + l2b[...]
            h = _dot(h, l3w[...]) + l3b[...]
            g_ids = jax.lax.broadcasted_iota(jnp.int32, (_G, _N), 0)
            starts = jnp.sum((batch_row[...] < g_ids).astype(jnp.int32),
                             axis=1, keepdims=True)
            n_ids = jax.lax.broadcasted_iota(jnp.int32, (_G, _N), 1)
            out = ob[...]
            for t in range(3):
                sel = (n_ids == starts + t).astype(jnp.float32)
                ft = _dot(sel, h)
                out = out + _dot(ft, ow[...][64 * t:64 * (t + 1), :])
            out_ref[...] = out


def _full(shape):
    return pl.BlockSpec(shape, lambda k: (0, 0))


def kernel(x, edge_index, edge_attr1, edge_attr2, batch, params):
    src = edge_index[0].reshape(_E, 1)
    dst = edge_index[1].reshape(1, _E)
    batch_row = batch.reshape(1, _N)
    h = x
    for (name, din, dout, kc) in _LAYERS:
        p = params[name]
        m = p["mlp"]
        w3c = m["W3"].reshape(64, din, dout).transpose(1, 0, 2).reshape(din, 64 * dout)
        nk = 64 // kc
        head = name == "conv_d"
        ops = [h, src, dst, edge_attr1, edge_attr2,
               m["W1"], m["b1"].reshape(1, 64), m["W2"], m["b2"].reshape(1, 64),
               w3c, m["b3"].reshape(din, dout), p["root"],
               p["bias"].reshape(1, dout)]
        in_specs = [
            _full((_N, din)), _full((_E, 1)), _full((1, _E)),
            _full((_E, _NEF)), _full((_E, _NEF)),
            _full((_NEF, 64)), _full((1, 64)), _full((64, 64)), _full((1, 64)),
            pl.BlockSpec((din, kc * dout), lambda k: (0, k)),
            _full((din, dout)), _full((din, dout)), _full((1, dout)),
        ]
        if head:
            ops += [batch_row, params["lin1W"], params["lin1b"].reshape(1, 128),
                    params["lin2W"], params["lin2b"].reshape(1, 64),
                    params["lin3W"], params["lin3b"].reshape(1, 64),
                    params["outW"], params["outb"].reshape(1, 1)]
            in_specs += [_full((1, _N)), _full((_D4, 128)), _full((1, 128)),
                         _full((128, 64)), _full((1, 64)), _full((64, 64)),
                         _full((1, 64)), _full((192, 1)), _full((1, 1))]
            out_shape = jax.ShapeDtypeStruct((_G, 1), jnp.float32)
            out_specs = _full((_G, 1))
        else:
            out_shape = jax.ShapeDtypeStruct((_N, dout), jnp.float32)
            out_specs = _full((_N, dout))
        h = pl.pallas_call(
            functools.partial(_conv_body, din, dout, kc, nk, head),
            grid=(nk,),
            in_specs=in_specs,
            out_specs=out_specs,
            out_shape=out_shape,
            scratch_shapes=[pltpu.VMEM((nk, _E, kc), jnp.float32),
                            pltpu.VMEM((_E, din), jnp.float32),
                            pltpu.VMEM((_E, dout), jnp.float32)],
        )(*ops)
    return h


# speedup vs baseline: 3.6928x; 2.6492x over previous
import functools

import jax
import jax.numpy as jnp
from jax.experimental import pallas as pl
from jax.experimental.pallas import tpu as pltpu

_DIN, _D1, _D2, _D3, _D4 = 4, 64, 128, 256, 64
_N, _E, _G, _NEF = 1024, 2048, 32, 3

_LAYERS = (
    ("conv_a", _DIN, _D1, 64),
    ("conv_b", _D1, _D2, 16),
    ("conv_c", _D2, _D3, 8),
    ("conv_d", _D3, _D4, 8),
)


def _dot(a, b):
    return jnp.dot(a, b, preferred_element_type=jnp.float32)


def _onehot_matmul(onehot, dense):
    hi = dense.astype(jnp.bfloat16).astype(jnp.float32)
    return _dot(onehot, hi) + _dot(onehot, dense - hi)


def _conv_body(din, dout, kc, nk, head, *refs):
    if head:
        (h_ref, src, dst, ea1, ea2, w1, b1, w2, b2, w3c, b3r, root, bias,
         batch_row, l1w, l1b, l2w, l2b, l3w, l3b, ow, ob, out_ref,
         h2_scr, xs_scr, msg_scr) = refs
    else:
        (h_ref, src, dst, ea1, ea2, w1, b1, w2, b2, w3c, b3r, root, bias,
         out_ref, h2_scr, xs_scr, msg_scr) = refs
    k = pl.program_id(0)

    @pl.when(k == 0)
    def _init():
        ea = ea1[...] + ea2[...]
        h1 = jnp.maximum(_dot(ea, w1[...]) + b1[...], 0.0)
        h2 = jnp.maximum(_dot(h1, w2[...]) + b2[...], 0.0)
        h2b = h2.astype(jnp.bfloat16)
        for i in range(nk):
            h2_scr[i] = h2b[:, i * kc:(i + 1) * kc]
        col_ids = jax.lax.broadcasted_iota(jnp.int32, (_E, _N), 1)
        gat = (src[...] == col_ids).astype(jnp.float32)
        xs = _onehot_matmul(gat, h_ref[...])
        xs_scr[...] = xs.astype(jnp.bfloat16)
        msg_scr[...] = _dot(xs, b3r[...])

    h2c = h2_scr[k].astype(jnp.float32)
    xs = xs_scr[...]
    u = _dot(xs, w3c[...])
    msg = msg_scr[...]
    for j in range(kc):
        msg = msg + h2c[:, j:j + 1] * u[:, j * dout:(j + 1) * dout]
    msg_scr[...] = msg

    @pl.when(k == nk - 1)
    def _fini():
        row_ids = jax.lax.broadcasted_iota(jnp.int32, (_N, _E), 0)
        sca = (dst[...] == row_ids).astype(jnp.float32)
        agg = _onehot_matmul(sca, msg_scr[...])
        h = jnp.maximum(agg + _dot(h_ref[...], root[...]) + bias[...], 0.0)
        if not head:
            out_ref[...] = h
        else:
            h = _dot(h, l1w[...]) + l1b[...]
            h = _dot(h, l2w[...]) + l2b[...]
            h = _dot(h, l3w[...]) + l3b[...]
            g_ids = jax.lax.broadcasted_iota(jnp.int32, (_G, _N), 0)
            starts = jnp.sum((batch_row[...] < g_ids).astype(jnp.int32),
                             axis=1, keepdims=True)
            n_ids = jax.lax.broadcasted_iota(jnp.int32, (_G, _N), 1)
            out = ob[...]
            for t in range(3):
                sel = (n_ids == starts + t).astype(jnp.float32)
                ft = _dot(sel, h)
                out = out + _dot(ft, ow[...][64 * t:64 * (t + 1), :])
            out_ref[...] = out


def _full(shape):
    return pl.BlockSpec(shape, lambda k: (0, 0))


def kernel(x, edge_index, edge_attr1, edge_attr2, batch, params):
    src = edge_index[0].reshape(_E, 1)
    dst = edge_index[1].reshape(1, _E)
    batch_row = batch.reshape(1, _N)
    h = x
    for (name, din, dout, kc) in _LAYERS:
        p = params[name]
        m = p["mlp"]
        w3c = m["W3"].reshape(64, din, dout).transpose(1, 0, 2)
        w3c = w3c.reshape(din, 64 * dout).astype(jnp.bfloat16)
        nk = 64 // kc
        head = name == "conv_d"
        ops = [h, src, dst, edge_attr1, edge_attr2,
               m["W1"], m["b1"].reshape(1, 64), m["W2"], m["b2"].reshape(1, 64),
               w3c, m["b3"].reshape(din, dout), p["root"],
               p["bias"].reshape(1, dout)]
        in_specs = [
            _full((_N, din)), _full((_E, 1)), _full((1, _E)),
            _full((_E, _NEF)), _full((_E, _NEF)),
            _full((_NEF, 64)), _full((1, 64)), _full((64, 64)), _full((1, 64)),
            pl.BlockSpec((din, kc * dout), lambda k: (0, k)),
            _full((din, dout)), _full((din, dout)), _full((1, dout)),
        ]
        if head:
            ops += [batch_row, params["lin1W"], params["lin1b"].reshape(1, 128),
                    params["lin2W"], params["lin2b"].reshape(1, 64),
                    params["lin3W"], params["lin3b"].reshape(1, 64),
                    params["outW"], params["outb"].reshape(1, 1)]
            in_specs += [_full((1, _N)), _full((_D4, 128)), _full((1, 128)),
                         _full((128, 64)), _full((1, 64)), _full((64, 64)),
                         _full((1, 64)), _full((192, 1)), _full((1, 1))]
            out_shape = jax.ShapeDtypeStruct((_G, 1), jnp.float32)
            out_specs = _full((_G, 1))
        else:
            out_shape = jax.ShapeDtypeStruct((_N, dout), jnp.float32)
            out_specs = _full((_N, dout))
        h = pl.pallas_call(
            functools.partial(_conv_body, din, dout, kc, nk, head),
            grid=(nk,),
            in_specs=in_specs,
            out_specs=out_specs,
            out_shape=out_shape,
            scratch_shapes=[pltpu.VMEM((nk, _E, kc), jnp.bfloat16),
                            pltpu.VMEM((_E, din), jnp.bfloat16),
                            pltpu.VMEM((_E, dout), jnp.float32)],
        )(*ops)
    return h
